# Initial kernel scaffold; baseline (speedup 1.0000x reference)
import jax
import jax.numpy as jnp
from jax.experimental import pallas as pl


def kernel(preds, inv_d, inv_t):
    raise NotImplementedError("write your pallas kernel here")


# speedup vs baseline: 8.4006x; 8.4006x over previous
import functools

import jax
import jax.numpy as jnp
from jax import lax
from jax.experimental import pallas as pl
from jax.experimental.pallas import tpu as pltpu

_AGENT_OFF = 1
_ACTION_OFF = 11
_LOC_OFF = 33
_NA, _NAC, _NL = 10, 22, 16


def _mask_kernel(inv_d_ref, inv_dT_ref, inv_t_ref, inv_tT_ref, wd_ref, wt_ref):
    nd = inv_d_ref.shape[0]
    nt = inv_t_ref.shape[0]
    ej = (inv_d_ref[:, 1:2]
          == lax.broadcasted_iota(jnp.int32, (nd, _NAC), 1)).astype(jnp.float32)
    eiT = (lax.broadcasted_iota(jnp.int32, (_NA, nd), 0)
           == inv_dT_ref[0:1, :]).astype(jnp.float32)
    wd_ref[...] = jnp.dot(eiT, ej, preferred_element_type=jnp.float32)
    tjk = inv_t_ref[:, 1:2] * _NL + inv_t_ref[:, 2:3]
    ejk = (tjk == lax.broadcasted_iota(jnp.int32, (nt, _NAC * _NL), 1)
           ).astype(jnp.float32)
    eiT_t = (lax.broadcasted_iota(jnp.int32, (_NA, nt), 0)
             == inv_tT_ref[0:1, :]).astype(jnp.float32)
    wt_ref[...] = jnp.dot(eiT_t, ejk, preferred_element_type=jnp.float32)


def _loss_kernel(pT_ref, wd_ref, wt_ref, out_ref, *, inv_nd, inv_nt):
    s = pl.program_id(0)
    p = pT_ref[...]
    r = p.shape[1]
    a = p[_AGENT_OFF:_AGENT_OFF + _NA, :]
    b = p[_ACTION_OFF:_ACTION_OFF + _NAC, :]
    c = p[_LOC_OFF:_LOC_OFF + _NL, :]
    accd = jnp.zeros((1, r), jnp.float32)
    acct = jnp.zeros((1, r), jnp.float32)
    for i in range(_NA):
        m_i = jnp.minimum(b, a[i:i + 1, :])
        accd += jnp.dot(wd_ref[i:i + 1, :], m_i,
                        preferred_element_type=jnp.float32)
        t_i = jnp.minimum(m_i[:, None, :], c[None, :, :])
        acct += jnp.dot(wt_ref[i:i + 1, :], t_i.reshape(_NAC * _NL, r),
                        preferred_element_type=jnp.float32)
    part = jnp.sum(accd * inv_nd + acct * inv_nt, keepdims=True)

    @pl.when(s == 0)
    def _init():
        out_ref[...] = jnp.zeros((1, 1), jnp.float32)

    out_ref[...] += part


def kernel(preds, inv_d, inv_t):
    preds = preds.astype(jnp.float32)
    inv_d = inv_d.astype(jnp.int32)
    inv_t = inv_t.astype(jnp.int32)
    n = preds.shape[0]
    nd, nt = inv_d.shape[0], inv_t.shape[0]

    wd, wt = pl.pallas_call(
        _mask_kernel,
        out_shape=(jax.ShapeDtypeStruct((_NA, _NAC), jnp.float32),
                   jax.ShapeDtypeStruct((_NA, _NAC * _NL), jnp.float32)),
    )(inv_d, inv_d.T, inv_t, inv_t.T)

    blk = 1024
    while n % blk:
        blk //= 2
    nsteps = n // blk
    pT = preds.T
    loss = pl.pallas_call(
        functools.partial(_loss_kernel,
                          inv_nd=1.0 / (n * nd), inv_nt=1.0 / (n * nt)),
        grid=(nsteps,),
        in_specs=[
            pl.BlockSpec((preds.shape[1], blk), lambda s: (0, s)),
            pl.BlockSpec((_NA, _NAC), lambda s: (0, 0)),
            pl.BlockSpec((_NA, _NAC * _NL), lambda s: (0, 0)),
        ],
        out_specs=pl.BlockSpec((1, 1), lambda s: (0, 0)),
        out_shape=jax.ShapeDtypeStruct((1, 1), jnp.float32),
    )(pT, wd, wt)
    return loss.reshape(1)


# speedup vs baseline: 8.4592x; 1.0070x over previous
import functools

import jax
import jax.numpy as jnp
from jax import lax
from jax.experimental import pallas as pl
from jax.experimental.pallas import tpu as pltpu

_AGENT_OFF = 1
_ACTION_OFF = 11
_LOC_OFF = 33
_NA, _NAC, _NL = 10, 22, 16
_NAC_P = 24


def _loss_kernel(p_ref, inv_d_ref, inv_t_ref, out_ref, wd_ref, wt_ref,
                 *, inv_nd, inv_nt):
    s = pl.program_id(0)

    @pl.when(s == 0)
    def _build_masks():
        nd = inv_d_ref.shape[0]
        nt = inv_t_ref.shape[0]
        ej = (inv_d_ref[:, 1:2]
              == lax.broadcasted_iota(jnp.int32, (nd, _NAC_P), 1)
              ).astype(jnp.float32)
        eiT = (lax.broadcasted_iota(jnp.int32, (_NA, nd), 0)
               == inv_d_ref[:, 0:1].T).astype(jnp.float32)
        wd_ref[...] = jnp.dot(eiT, ej, preferred_element_type=jnp.float32)
        tkj = inv_t_ref[:, 2:3] * _NAC_P + inv_t_ref[:, 1:2]
        ekj = (tkj == lax.broadcasted_iota(jnp.int32, (nt, _NL * _NAC_P), 1)
               ).astype(jnp.float32)
        eiT_t = (lax.broadcasted_iota(jnp.int32, (_NA, nt), 0)
                 == inv_t_ref[:, 0:1].T).astype(jnp.float32)
        wt_ref[...] = jnp.dot(eiT_t, ekj, preferred_element_type=jnp.float32)
        out_ref[...] = jnp.zeros((1, 1), jnp.float32)

    p = p_ref[...].T
    r = p.shape[1]
    a = p[_AGENT_OFF:_AGENT_OFF + _NA, :]
    b = p[_ACTION_OFF:_ACTION_OFF + _NAC_P, :]
    c = p[_LOC_OFF:_LOC_OFF + _NL, :]
    cb = jnp.broadcast_to(c[:, None, :], (_NL, _NAC_P, r))
    accd = jnp.zeros((1, r), jnp.float32)
    acct = jnp.zeros((1, r), jnp.float32)
    for i in range(_NA):
        m_i = jnp.minimum(b, a[i:i + 1, :])
        accd += jnp.dot(wd_ref[i:i + 1, :], m_i,
                        preferred_element_type=jnp.float32)
        t_i = jnp.minimum(cb, m_i[None, :, :])
        acct += jnp.dot(wt_ref[i:i + 1, :], t_i.reshape(_NL * _NAC_P, r),
                        preferred_element_type=jnp.float32)
    part = jnp.sum(accd * inv_nd + acct * inv_nt, keepdims=True)
    out_ref[...] += part


def kernel(preds, inv_d, inv_t):
    preds = preds.astype(jnp.float32)
    inv_d = inv_d.astype(jnp.int32)
    inv_t = inv_t.astype(jnp.int32)
    n, ncols = preds.shape
    nd, nt = inv_d.shape[0], inv_t.shape[0]

    blk = 1024
    while n % blk:
        blk //= 2
    nsteps = n // blk
    loss = pl.pallas_call(
        functools.partial(_loss_kernel,
                          inv_nd=1.0 / (n * nd), inv_nt=1.0 / (n * nt)),
        grid=(nsteps,),
        in_specs=[
            pl.BlockSpec((blk, ncols), lambda s: (s, 0)),
            pl.BlockSpec(inv_d.shape, lambda s: (0, 0)),
            pl.BlockSpec(inv_t.shape, lambda s: (0, 0)),
        ],
        out_specs=pl.BlockSpec((1, 1), lambda s: (0, 0)),
        out_shape=jax.ShapeDtypeStruct((1, 1), jnp.float32),
        scratch_shapes=[pltpu.VMEM((_NA, _NAC_P), jnp.float32),
                        pltpu.VMEM((_NA, _NL * _NAC_P), jnp.float32)],
    )(preds, inv_d, inv_t)
    return loss.reshape(1)


# speedup vs baseline: 10.0901x; 1.1928x over previous
import functools

import jax
import jax.numpy as jnp
from jax import lax
from jax.experimental import pallas as pl
from jax.experimental.pallas import tpu as pltpu

_AGENT_OFF = 1
_ACTION_OFF = 11
_LOC_OFF = 33
_NA, _NAC, _NL = 10, 22, 16
_NAC_P = 32


def _loss_kernel(p_ref, inv_d_ref, inv_t_ref, out_ref, wd_ref, wt_ref,
                 *, inv_nd, inv_nt):
    s = pl.program_id(0)

    @pl.when(s == 0)
    def _build_masks():
        nd = inv_d_ref.shape[0]
        nt = inv_t_ref.shape[0]
        dij = inv_d_ref[:, 0:1] * _NAC_P + inv_d_ref[:, 1:2]
        e_d = (dij == lax.broadcasted_iota(jnp.int32, (nd, _NA * _NAC_P), 1)
               ).astype(jnp.float32)
        wd = jnp.dot(jnp.full((1, nd), 1.0, jnp.float32), e_d,
                     preferred_element_type=jnp.float32)
        wd_ref[...] = wd.astype(jnp.bfloat16)
        tij = inv_t_ref[:, 0:1] * _NAC_P + inv_t_ref[:, 1:2]
        e_ij = (tij == lax.broadcasted_iota(jnp.int32, (nt, _NA * _NAC_P), 1)
                ).astype(jnp.float32)
        ekT = (lax.broadcasted_iota(jnp.int32, (_NL, nt), 0)
               == inv_t_ref[:, 2:3].T).astype(jnp.float32)
        wt = jnp.dot(ekT, e_ij, preferred_element_type=jnp.float32)
        wt_ref[...] = wt.astype(jnp.bfloat16)
        out_ref[...] = jnp.zeros((1, 1), jnp.float32)

    p = p_ref[...].T.astype(jnp.bfloat16)
    r = p.shape[1]
    a = p[_AGENT_OFF:_AGENT_OFF + _NA, :]
    b = p[_ACTION_OFF:_ACTION_OFF + _NAC_P, :]
    c = p[_LOC_OFF:_LOC_OFF + _NL, :]
    m = jnp.minimum(a[:, None, :], b[None, :, :])
    m = m.reshape(_NA * _NAC_P, r)
    accd = jnp.dot(wd_ref[...], m, preferred_element_type=jnp.float32)
    acct = jnp.zeros((1, r), jnp.float32)
    for k in range(_NL):
        t_k = jnp.minimum(m, c[k:k + 1, :])
        acct += jnp.dot(wt_ref[k:k + 1, :], t_k,
                        preferred_element_type=jnp.float32)
    part = jnp.sum(accd * inv_nd + acct * inv_nt, keepdims=True)
    out_ref[...] += part


def kernel(preds, inv_d, inv_t):
    preds = preds.astype(jnp.float32)
    inv_d = inv_d.astype(jnp.int32)
    inv_t = inv_t.astype(jnp.int32)
    n, ncols = preds.shape
    nd, nt = inv_d.shape[0], inv_t.shape[0]

    blk = 2048
    while n % blk:
        blk //= 2
    nsteps = n // blk
    loss = pl.pallas_call(
        functools.partial(_loss_kernel,
                          inv_nd=1.0 / (n * nd), inv_nt=1.0 / (n * nt)),
        grid=(nsteps,),
        in_specs=[
            pl.BlockSpec((blk, ncols), lambda s: (s, 0)),
            pl.BlockSpec(inv_d.shape, lambda s: (0, 0)),
            pl.BlockSpec(inv_t.shape, lambda s: (0, 0)),
        ],
        out_specs=pl.BlockSpec((1, 1), lambda s: (0, 0)),
        out_shape=jax.ShapeDtypeStruct((1, 1), jnp.float32),
        scratch_shapes=[pltpu.VMEM((1, _NA * _NAC_P), jnp.bfloat16),
                        pltpu.VMEM((_NL, _NA * _NAC_P), jnp.bfloat16)],
    )(preds, inv_d, inv_t)
    return loss.reshape(1)


# speedup vs baseline: 13.5835x; 1.3462x over previous
import functools

import jax
import jax.numpy as jnp
from jax import lax
from jax.experimental import pallas as pl
from jax.experimental.pallas import tpu as pltpu

_AGENT_OFF = 1
_ACTION_OFF = 11
_LOC_OFF = 33
_NA, _NAC, _NL = 10, 22, 16
_NAC_P = 32
_NIJ = _NA * _NAC_P


def _loss_kernel(p_ref, inv_d_ref, inv_t_ref, out_ref,
                 wd_ref, u_ref, vm_ref, vc_ref, *, inv_nd, inv_nt, n_valid):
    s = pl.program_id(0)

    @pl.when(s == 0)
    def _build_masks():
        nd = inv_d_ref.shape[0]
        nt = inv_t_ref.shape[0]
        dij = inv_d_ref[:, 0:1] * _NAC_P + inv_d_ref[:, 1:2]
        e_d = (dij == lax.broadcasted_iota(jnp.int32, (nd, _NIJ), 1)
               ).astype(jnp.float32)
        wd = jnp.dot(jnp.full((1, nd), 1.0, jnp.float32), e_d,
                     preferred_element_type=jnp.float32)
        wd_ref[...] = wd.astype(jnp.bfloat16)
        col = lax.broadcasted_iota(jnp.int32, (1, _NIJ), 1)
        u_row = (col % _NAC_P < _NAC).astype(jnp.float32)
        u_ref[...] = u_row.astype(jnp.bfloat16)
        tij = inv_t_ref[:, 0:1] * _NAC_P + inv_t_ref[:, 1:2]
        e_ij = (tij == lax.broadcasted_iota(jnp.int32, (nt, _NIJ), 1)
                ).astype(jnp.float32)
        ekT = (lax.broadcasted_iota(jnp.int32, (_NL, nt), 0)
               == inv_t_ref[:, 2:3].T).astype(jnp.float32)
        wt = jnp.dot(ekT, e_ij, preferred_element_type=jnp.float32)
        v = jnp.broadcast_to(u_row, (_NL, _NIJ)) - wt
        flat = (lax.broadcasted_iota(jnp.int32, (_NL, _NIJ), 0) * _NIJ
                + lax.broadcasted_iota(jnp.int32, (_NL, _NIJ), 1)
                ).astype(jnp.float32)
        score = v * (flat + 1.0)
        ij_iota = lax.broadcasted_iota(jnp.int32, (1, _NIJ), 1
                                       ).astype(jnp.float32)
        k_iota = lax.broadcasted_iota(jnp.int32, (1, _NL), 1
                                      ).astype(jnp.float32)
        for t in range(n_valid):
            pos = jnp.max(score) - 1.0
            kk = jnp.floor((pos + 0.5) / _NIJ)
            ij = pos - kk * _NIJ
            vm_ref[t:t + 1, :] = (ij_iota == ij).astype(jnp.bfloat16)
            vc_ref[t:t + 1, :] = (k_iota == kk).astype(jnp.bfloat16)
            score = score * (1.0 - (flat == pos).astype(jnp.float32))
        out_ref[...] = jnp.zeros((1, 1), jnp.float32)

    p = p_ref[...].T.astype(jnp.bfloat16)
    r = p.shape[1]
    a = p[_AGENT_OFF:_AGENT_OFF + _NA, :]
    b = p[_ACTION_OFF:_ACTION_OFF + _NAC_P, :]
    c = p[_LOC_OFF:_LOC_OFF + _NL, :]
    m = jnp.minimum(a[:, None, :], b[None, :, :])
    m = m.reshape(_NIJ, r)
    accd = jnp.dot(wd_ref[...], m, preferred_element_type=jnp.float32)
    acc = jnp.minimum(m, c[0:1, :])
    for k in range(1, _NL):
        acc += jnp.minimum(m, c[k:k + 1, :])
    acct = jnp.dot(u_ref[...], acc, preferred_element_type=jnp.float32)
    if n_valid:
        mm = jnp.dot(vm_ref[...], m, preferred_element_type=jnp.float32)
        cc = jnp.dot(vc_ref[...], c, preferred_element_type=jnp.float32)
        acct -= jnp.sum(jnp.minimum(mm, cc), axis=0, keepdims=True)
    part = jnp.sum(accd * inv_nd + acct * inv_nt, keepdims=True)
    out_ref[...] += part


def kernel(preds, inv_d, inv_t):
    preds = preds.astype(jnp.float32)
    inv_d = inv_d.astype(jnp.int32)
    inv_t = inv_t.astype(jnp.int32)
    n, ncols = preds.shape
    nd, nt = inv_d.shape[0], inv_t.shape[0]
    n_valid = _NA * _NAC * _NL - nt

    blk = 2048
    while n % blk:
        blk //= 2
    nsteps = n // blk
    loss = pl.pallas_call(
        functools.partial(_loss_kernel, inv_nd=1.0 / (n * nd),
                          inv_nt=1.0 / (n * nt), n_valid=n_valid),
        grid=(nsteps,),
        in_specs=[
            pl.BlockSpec((blk, ncols), lambda s: (s, 0)),
            pl.BlockSpec(inv_d.shape, lambda s: (0, 0)),
            pl.BlockSpec(inv_t.shape, lambda s: (0, 0)),
        ],
        out_specs=pl.BlockSpec((1, 1), lambda s: (0, 0)),
        out_shape=jax.ShapeDtypeStruct((1, 1), jnp.float32),
        scratch_shapes=[pltpu.VMEM((1, _NIJ), jnp.bfloat16),
                        pltpu.VMEM((1, _NIJ), jnp.bfloat16),
                        pltpu.VMEM((max(n_valid, 1), _NIJ), jnp.bfloat16),
                        pltpu.VMEM((max(n_valid, 1), _NL), jnp.bfloat16)],
    )(preds, inv_d, inv_t)
    return loss.reshape(1)


# speedup vs baseline: 13.7429x; 1.0117x over previous
import functools

import jax
import jax.numpy as jnp
from jax import lax
from jax.experimental import pallas as pl
from jax.experimental.pallas import tpu as pltpu

_AGENT_OFF = 1
_ACTION_OFF = 11
_LOC_OFF = 33
_NA, _NAC, _NL = 10, 22, 16
_NAC_P = 32
_NIJ = _NA * _NAC_P


def _loss_kernel(p_ref, inv_d_ref, inv_t_ref, out_ref,
                 wd_ref, u_ref, vm_ref, vc_ref, acc_ref,
                 *, inv_nd, inv_nt, n_valid):
    s = pl.program_id(0)

    @pl.when(s == 0)
    def _build_masks():
        nd = inv_d_ref.shape[0]
        nt = inv_t_ref.shape[0]
        dij = inv_d_ref[:, 0:1] * _NAC_P + inv_d_ref[:, 1:2]
        e_d = (dij == lax.broadcasted_iota(jnp.int32, (nd, _NIJ), 1)
               ).astype(jnp.float32)
        wd = jnp.dot(jnp.full((1, nd), 1.0, jnp.float32), e_d,
                     preferred_element_type=jnp.float32)
        wd_ref[...] = wd.astype(jnp.bfloat16)
        col = lax.broadcasted_iota(jnp.int32, (1, _NIJ), 1)
        u_row = (col % _NAC_P < _NAC).astype(jnp.float32)
        u_ref[...] = u_row.astype(jnp.bfloat16)
        tij = inv_t_ref[:, 0:1] * _NAC_P + inv_t_ref[:, 1:2]
        e_ij = (tij == lax.broadcasted_iota(jnp.int32, (nt, _NIJ), 1)
                ).astype(jnp.float32)
        ekT = (lax.broadcasted_iota(jnp.int32, (_NL, nt), 0)
               == inv_t_ref[:, 2:3].T).astype(jnp.float32)
        wt = jnp.dot(ekT, e_ij, preferred_element_type=jnp.float32)
        v = jnp.broadcast_to(u_row, (_NL, _NIJ)) - wt
        flat = (lax.broadcasted_iota(jnp.int32, (_NL, _NIJ), 0) * _NIJ
                + lax.broadcasted_iota(jnp.int32, (_NL, _NIJ), 1)
                ).astype(jnp.float32)
        score = v * (flat + 1.0)
        ij_iota = lax.broadcasted_iota(jnp.int32, (1, _NIJ), 1
                                       ).astype(jnp.float32)
        k_iota = lax.broadcasted_iota(jnp.int32, (1, _NL), 1
                                      ).astype(jnp.float32)
        for t in range(n_valid):
            pos = jnp.max(score) - 1.0
            kk = jnp.floor((pos + 0.5) / _NIJ)
            ij = pos - kk * _NIJ
            vm_ref[t:t + 1, :] = (ij_iota == ij).astype(jnp.bfloat16)
            vc_ref[t:t + 1, :] = (k_iota == kk).astype(jnp.bfloat16)
            score = score * (1.0 - (flat == pos).astype(jnp.float32))
        out_ref[...] = jnp.zeros((1, 1), jnp.float32)

    p = p_ref[...].astype(jnp.bfloat16).T
    r = p.shape[1]
    a = p[_AGENT_OFF:_AGENT_OFF + _NA, :]
    b = p[_ACTION_OFF:_ACTION_OFF + _NAC_P, :]
    c = p[_LOC_OFF:_LOC_OFF + _NL, :]
    m = jnp.minimum(a[:, None, :], b[None, :, :])
    m = m.reshape(_NIJ, r)
    accd = jnp.dot(wd_ref[...], m, preferred_element_type=jnp.float32)
    ch = 64
    for lo in range(0, _NIJ, ch):
        mc = m[lo:lo + ch, :]
        acc_c = jnp.minimum(mc, c[0:1, :])
        for k in range(1, _NL):
            acc_c += jnp.minimum(mc, c[k:k + 1, :])
        acc_ref[lo:lo + ch, :] = acc_c
    acct = jnp.dot(u_ref[...], acc_ref[...], preferred_element_type=jnp.float32)
    if n_valid:
        mm = jnp.dot(vm_ref[...], m, preferred_element_type=jnp.float32)
        cc = jnp.dot(vc_ref[...], c, preferred_element_type=jnp.float32)
        acct -= jnp.sum(jnp.minimum(mm, cc), axis=0, keepdims=True)
    part = jnp.sum(accd * inv_nd + acct * inv_nt, keepdims=True)
    out_ref[...] += part


def kernel(preds, inv_d, inv_t):
    preds = preds.astype(jnp.float32)
    inv_d = inv_d.astype(jnp.int32)
    inv_t = inv_t.astype(jnp.int32)
    n, ncols = preds.shape
    nd, nt = inv_d.shape[0], inv_t.shape[0]
    n_valid = _NA * _NAC * _NL - nt

    blk = 2048
    while n % blk:
        blk //= 2
    nsteps = n // blk
    loss = pl.pallas_call(
        functools.partial(_loss_kernel, inv_nd=1.0 / (n * nd),
                          inv_nt=1.0 / (n * nt), n_valid=n_valid),
        grid=(nsteps,),
        in_specs=[
            pl.BlockSpec((blk, ncols), lambda s: (s, 0)),
            pl.BlockSpec(inv_d.shape, lambda s: (0, 0)),
            pl.BlockSpec(inv_t.shape, lambda s: (0, 0)),
        ],
        out_specs=pl.BlockSpec((1, 1), lambda s: (0, 0)),
        out_shape=jax.ShapeDtypeStruct((1, 1), jnp.float32),
        scratch_shapes=[pltpu.VMEM((1, _NIJ), jnp.bfloat16),
                        pltpu.VMEM((1, _NIJ), jnp.bfloat16),
                        pltpu.VMEM((max(n_valid, 1), _NIJ), jnp.bfloat16),
                        pltpu.VMEM((max(n_valid, 1), _NL), jnp.bfloat16),
                        pltpu.VMEM((_NIJ, blk), jnp.bfloat16)],
    )(preds, inv_d, inv_t)
    return loss.reshape(1)
